# Initial kernel scaffold; baseline (speedup 1.0000x reference)
import jax
import jax.numpy as jnp
from jax.experimental import pallas as pl


def kernel(p, x, o, P):
    raise NotImplementedError("write your pallas kernel here")


# speedup vs baseline: 11.8026x; 11.8026x over previous
import functools

import jax
import jax.numpy as jnp
from jax import lax
from jax.experimental import pallas as pl
from jax.experimental.pallas import tpu as pltpu
from jax.experimental.pallas import tpu_sc as plsc

EPS_BN = 1e-5
STRIDE = 4
KNN_K = 16
BN_ROWS = 2048


def _fps_body(px_ref, py_ref, pz_ref, ox_ref, oy_ref, oz_ref, *, n, m):
    S, L = px_ref.shape
    MS, ML = ox_ref.shape
    px = px_ref[...]
    py = py_ref[...]
    pz = pz_ref[...]
    iota = (lax.broadcasted_iota(jnp.int32, (S, L), 0) * L
            + lax.broadcasted_iota(jnp.int32, (S, L), 1))
    miota = (lax.broadcasted_iota(jnp.int32, (MS, ML), 0) * ML
             + lax.broadcasted_iota(jnp.int32, (MS, ML), 1))
    l0x = px_ref[0, 0]
    l0y = py_ref[0, 0]
    l0z = pz_ref[0, 0]
    ox_ref[...] = jnp.where(miota == 0, l0x, 0.0)
    oy_ref[...] = jnp.where(miota == 0, l0y, 0.0)
    oz_ref[...] = jnp.where(miota == 0, l0z, 0.0)

    def body(i, carry):
        dists, lx, ly, lz = carry
        d = (px - lx) ** 2 + (py - ly) ** 2 + (pz - lz) ** 2
        dists = jnp.minimum(dists, d)
        gmax = jnp.max(dists)
        idx = jnp.min(jnp.where(dists >= gmax, iota, n))
        sel = iota == idx
        nlx = jnp.sum(jnp.where(sel, px, 0.0))
        nly = jnp.sum(jnp.where(sel, py, 0.0))
        nlz = jnp.sum(jnp.where(sel, pz, 0.0))
        oh = miota == i
        ox_ref[...] += jnp.where(oh, nlx, 0.0)
        oy_ref[...] += jnp.where(oh, nly, 0.0)
        oz_ref[...] += jnp.where(oh, nlz, 0.0)
        return dists, nlx, nly, nlz

    d0 = jnp.full((S, L), 1e10, jnp.float32)
    lax.fori_loop(1, m, body, (d0, l0x, l0y, l0z))


def _fps(px, py, pz, m):
    S, L = px.shape
    n = S * L
    MS, ML = (m // 128, 128) if m >= 128 else (1, m)
    return pl.pallas_call(
        functools.partial(_fps_body, n=n, m=m),
        out_shape=[jax.ShapeDtypeStruct((MS, ML), jnp.float32)] * 3,
    )(px, py, pz)


def _knn_body(q_ref, rT_ref, idx_ref, d2_ref, *, K, n, want_d2):
    q = q_ref[...]
    rT = rT_ref[...]
    qn = jnp.sum(q * q, axis=1, keepdims=True)
    rn = jnp.sum(rT * rT, axis=0, keepdims=True)
    d2 = qn + rn - 2.0 * jnp.dot(q, rT, preferred_element_type=jnp.float32)
    Bq = q.shape[0]
    col = lax.broadcasted_iota(jnp.int32, (Bq, n), 1)
    if want_d2:
        d2e = ((q[:, 0:1] - rT[0:1, :]) ** 2 + (q[:, 1:2] - rT[1:2, :]) ** 2
               + (q[:, 2:3] - rT[2:3, :]) ** 2)
    for k in range(K):
        rowmin = jnp.min(d2, axis=1, keepdims=True)
        amin = jnp.min(jnp.where(d2 <= rowmin, col, n), axis=1, keepdims=True)
        idx_ref[:, k:k + 1] = amin
        if want_d2:
            d2_ref[:, k:k + 1] = jnp.min(
                jnp.where(col == amin, d2e, jnp.inf), axis=1, keepdims=True)
        d2 = jnp.where(col == amin, jnp.inf, d2)


def _knn(q, rT, K, want_d2):
    m = q.shape[0]
    n = rT.shape[1]
    Bq = min(256, m)
    kfn = functools.partial(_knn_body, K=K, n=n, want_d2=want_d2)
    idx, d2 = pl.pallas_call(
        kfn,
        grid=(m // Bq,),
        in_specs=[pl.BlockSpec((Bq, 3), lambda i: (i, 0)),
                  pl.BlockSpec((3, n), lambda i: (0, 0))],
        out_specs=[pl.BlockSpec((Bq, K), lambda i: (i, 0)),
                   pl.BlockSpec((Bq, K), lambda i: (i, 0))],
        out_shape=[jax.ShapeDtypeStruct((m, K), jnp.int32),
                   jax.ShapeDtypeStruct((m, K), jnp.float32)],
    )(q, rT)
    return idx, d2


def _sc_gather(table, idx):
    V, D = table.shape
    B = idx.shape[0]
    info = plsc.get_sparse_core_info()
    NW = info.num_cores * info.num_subcores
    assert B % NW == 0, (B, NW)
    b_per_w = B // NW
    if b_per_w <= 128:
        assert b_per_w % 8 == 0, b_per_w
        ch = b_per_w
    else:
        assert b_per_w % 128 == 0, b_per_w
        ch = 128
    nchunks = b_per_w // ch
    mesh = plsc.VectorSubcoreMesh(core_axis_name="c", subcore_axis_name="s")

    @functools.partial(
        pl.kernel,
        mesh=mesh,
        compiler_params=pltpu.CompilerParams(use_tc_tiling_on_sc=False),
        out_type=jax.ShapeDtypeStruct((B, D), jnp.float32),
        scratch_types=[
            pltpu.VMEM((b_per_w,), jnp.int32),
            pltpu.VMEM((b_per_w, D), jnp.float32),
            pltpu.SemaphoreType.DMA,
        ],
    )
    def gk(table_hbm, idx_hbm, out_hbm, idx_v, rows_v, sem):
        wid = lax.axis_index("s") * info.num_cores + lax.axis_index("c")
        base = wid * b_per_w
        pltpu.sync_copy(idx_hbm.at[pl.ds(base, b_per_w)], idx_v)
        copies = []
        for c in range(nchunks):
            copies.append(pltpu.async_copy(
                table_hbm.at[idx_v.at[pl.ds(c * ch, ch)]],
                rows_v.at[pl.ds(c * ch, ch)], sem))
        for cp in copies:
            cp.wait()
        pltpu.sync_copy(rows_v, out_hbm.at[pl.ds(base, b_per_w)])

    return gk(table, idx)


def _xwt(x, w):
    return lax.dot_general(x, w, (((1,), (1,)), ((), ())),
                           preferred_element_type=jnp.float32)


def _lin_stats_body(x_ref, w_ref, b_ref, h_ref, ssum_ref, ssq_ref):
    h = _xwt(x_ref[...], w_ref[...]) + b_ref[...]
    h_ref[...] = h

    @pl.when(pl.program_id(0) == 0)
    def _():
        ssum_ref[...] = jnp.zeros_like(ssum_ref)
        ssq_ref[...] = jnp.zeros_like(ssq_ref)

    ssum_ref[...] += jnp.sum(h, axis=0, keepdims=True)
    ssq_ref[...] += jnp.sum(h * h, axis=0, keepdims=True)


def _lin_stats(x, w, b):
    n, cin = x.shape
    cout = w.shape[0]
    Bn = min(BN_ROWS, n)
    return pl.pallas_call(
        _lin_stats_body,
        grid=(n // Bn,),
        in_specs=[pl.BlockSpec((Bn, cin), lambda i: (i, 0)),
                  pl.BlockSpec((cout, cin), lambda i: (0, 0)),
                  pl.BlockSpec((1, cout), lambda i: (0, 0))],
        out_specs=[pl.BlockSpec((Bn, cout), lambda i: (i, 0)),
                   pl.BlockSpec((1, cout), lambda i: (0, 0)),
                   pl.BlockSpec((1, cout), lambda i: (0, 0))],
        out_shape=[jax.ShapeDtypeStruct((n, cout), jnp.float32),
                   jax.ShapeDtypeStruct((1, cout), jnp.float32),
                   jax.ShapeDtypeStruct((1, cout), jnp.float32)],
    )(x, w, b)


def _bn_scale_shift(ssum, ssq, g, be, cnt):
    mu = ssum / cnt
    var = jnp.maximum(ssq / cnt - mu * mu, 0.0)
    scale = g / jnp.sqrt(var + EPS_BN)
    return scale, be - mu * scale


def _norm_body(h_ref, ssum_ref, ssq_ref, g_ref, be_ref, o_ref, *, cnt):
    scale, shift = _bn_scale_shift(ssum_ref[...], ssq_ref[...], g_ref[...],
                                   be_ref[...], cnt)
    o_ref[...] = jnp.maximum(h_ref[...] * scale + shift, 0.0)


def _norm_relu(h, ssum, ssq, g, be):
    n, c = h.shape
    Bn = min(BN_ROWS, n)
    return pl.pallas_call(
        functools.partial(_norm_body, cnt=float(n)),
        grid=(n // Bn,),
        in_specs=[pl.BlockSpec((Bn, c), lambda i: (i, 0)),
                  pl.BlockSpec((1, c), lambda i: (0, 0)),
                  pl.BlockSpec((1, c), lambda i: (0, 0)),
                  pl.BlockSpec((1, c), lambda i: (0, 0)),
                  pl.BlockSpec((1, c), lambda i: (0, 0))],
        out_specs=pl.BlockSpec((Bn, c), lambda i: (i, 0)),
        out_shape=jax.ShapeDtypeStruct((n, c), jnp.float32),
    )(h, ssum, ssq, g, be)


def _norm_head_body(h_ref, ssum_ref, ssq_ref, g_ref, be_ref, w2_ref, b2_ref,
                    o_ref, *, cnt):
    scale, shift = _bn_scale_shift(ssum_ref[...], ssq_ref[...], g_ref[...],
                                   be_ref[...], cnt)
    y = jnp.maximum(h_ref[...] * scale + shift, 0.0)
    o_ref[...] = _xwt(y, w2_ref[...]) + b2_ref[...]


def _norm_head(h, ssum, ssq, g, be, w2, b2):
    n, c = h.shape
    c2 = w2.shape[0]
    Bn = min(BN_ROWS, n)
    return pl.pallas_call(
        functools.partial(_norm_head_body, cnt=float(n)),
        grid=(n // Bn,),
        in_specs=[pl.BlockSpec((Bn, c), lambda i: (i, 0)),
                  pl.BlockSpec((1, c), lambda i: (0, 0)),
                  pl.BlockSpec((1, c), lambda i: (0, 0)),
                  pl.BlockSpec((1, c), lambda i: (0, 0)),
                  pl.BlockSpec((1, c), lambda i: (0, 0)),
                  pl.BlockSpec((c2, c), lambda i: (0, 0)),
                  pl.BlockSpec((1, c2), lambda i: (0, 0))],
        out_specs=pl.BlockSpec((Bn, c2), lambda i: (i, 0)),
        out_shape=jax.ShapeDtypeStruct((n, c2), jnp.float32),
    )(h, ssum, ssq, g, be, w2, b2)


def _bn_relu_lin(x, w, b, g, be):
    h, ssum, ssq = _lin_stats(x, w, b)
    return _norm_relu(h, ssum, ssq, g, be)


def _enc_stats_body(G_ref, np_ref, wp_ref, ssum_ref, ssq_ref, *, K):
    npp = np_ref[...]
    wp = wp_ref[...]

    @pl.when(pl.program_id(0) == 0)
    def _():
        ssum_ref[...] = jnp.zeros_like(ssum_ref)
        ssq_ref[...] = jnp.zeros_like(ssq_ref)

    cout = wp.shape[0]
    psum = jnp.zeros((1, cout), jnp.float32)
    psq = jnp.zeros((1, cout), jnp.float32)
    for k in range(K):
        hk = _xwt(G_ref[k] - npp, wp)
        psum = psum + jnp.sum(hk, axis=0, keepdims=True)
        psq = psq + jnp.sum(hk * hk, axis=0, keepdims=True)
    ssum_ref[...] += psum
    ssq_ref[...] += psq


def _enc_max_body(G_ref, np_ref, wp_ref, ssum_ref, ssq_ref, g_ref,
                  b_ref, o_ref, *, K, cnt):
    npp = np_ref[...]
    wp = wp_ref[...]
    scale, shift = _bn_scale_shift(ssum_ref[...], ssq_ref[...], g_ref[...],
                                   b_ref[...], cnt)
    Bm = np_ref.shape[0]
    cout = wp.shape[0]
    acc = jnp.full((Bm, cout), -jnp.inf, jnp.float32)
    for k in range(K):
        hk = _xwt(G_ref[k] - npp, wp)
        acc = jnp.maximum(acc, jnp.maximum(hk * scale + shift, 0.0))
    o_ref[...] = acc


def _enc_level(G3, np_pad, wp, g, b, K, m):
    cout, dpad = wp.shape
    Bm = min(512, m)
    grid = (m // Bm,)
    gspec = pl.BlockSpec((K, Bm, dpad), lambda i: (0, i, 0))
    npspec = pl.BlockSpec((Bm, dpad), lambda i: (i, 0))
    wpspec = pl.BlockSpec((cout, dpad), lambda i: (0, 0))
    cspec = pl.BlockSpec((1, cout), lambda i: (0, 0))
    ssum, ssq = pl.pallas_call(
        functools.partial(_enc_stats_body, K=K),
        grid=grid,
        in_specs=[gspec, npspec, wpspec],
        out_specs=[cspec, cspec],
        out_shape=[jax.ShapeDtypeStruct((1, cout), jnp.float32)] * 2,
    )(G3, np_pad, wp)
    return pl.pallas_call(
        functools.partial(_enc_max_body, K=K, cnt=float(m * K)),
        grid=grid,
        in_specs=[gspec, npspec, wpspec, cspec, cspec, cspec, cspec],
        out_specs=pl.BlockSpec((Bm, cout), lambda i: (i, 0)),
        out_shape=jax.ShapeDtypeStruct((m, cout), jnp.float32),
    )(G3, np_pad, wp, ssum, ssq, g, b)


def _mid_body(x5_ref, w2_ref, b2_ref, wa_ref, wb_ref, b1_ref, g1_ref,
              be1_ref, o_ref):
    x5 = x5_ref[...]
    g5 = jnp.maximum(_xwt(jnp.mean(x5, axis=0, keepdims=True), w2_ref[...])
                     + b2_ref[...], 0.0)
    h = _xwt(x5, wa_ref[...]) + _xwt(g5, wb_ref[...]) + b1_ref[...]
    mu = jnp.mean(h, axis=0, keepdims=True)
    var = jnp.mean((h - mu) ** 2, axis=0, keepdims=True)
    o_ref[...] = jnp.maximum((h - mu) / jnp.sqrt(var + EPS_BN) * g1_ref[...]
                             + be1_ref[...], 0.0)


def _dec_comb_body(a_ref, G_ref, d2_ref, o_ref):
    rec = 1.0 / (d2_ref[...] + 1e-8)
    w = rec / jnp.sum(rec, axis=1, keepdims=True)
    acc = a_ref[...]
    for j in range(3):
        acc = acc + G_ref[j] * w[:, j:j + 1]
    o_ref[...] = acc


def _dec_combine(a, G3, d2):
    mf, c = a.shape
    Bm = min(BN_ROWS, mf)
    return pl.pallas_call(
        _dec_comb_body,
        grid=(mf // Bm,),
        in_specs=[pl.BlockSpec((Bm, c), lambda i: (i, 0)),
                  pl.BlockSpec((3, Bm, c), lambda i: (0, i, 0)),
                  pl.BlockSpec((Bm, 3), lambda i: (i, 0))],
        out_specs=pl.BlockSpec((Bm, c), lambda i: (i, 0)),
        out_shape=jax.ShapeDtypeStruct((mf, c), jnp.float32),
    )(a, G3, d2)


def _pad_cols(a, d):
    c = a.shape[1]
    return a if c == d else jnp.pad(a, ((0, 0), (0, d - c)))


def _row(v):
    return v.reshape(1, -1)


def kernel(p, x, o, P):
    del o
    n0 = p.shape[0]

    coords = [(p[:, 0].reshape(n0 // 128, 128),
               p[:, 1].reshape(n0 // 128, 128),
               p[:, 2].reshape(n0 // 128, 128))]
    sizes = [n0]
    for _ in range(4):
        m = sizes[-1] // STRIDE
        cx, cy, cz = coords[-1]
        ox, oy, oz = _fps(cx, cy, cz, m)
        MS, ML = (m // 128, 128) if m >= 128 else (1, m)
        coords.append((ox.reshape(MS, ML), oy.reshape(MS, ML),
                       oz.reshape(MS, ML)))
        sizes.append(m)

    pts = []
    ptsT = []
    for (cx, cy, cz), m in zip(coords, sizes):
        fx, fy, fz = cx.reshape(-1), cy.reshape(-1), cz.reshape(-1)
        pts.append(jnp.stack([fx, fy, fz], axis=1))
        ptsT.append(jnp.stack([fx, fy, fz], axis=0))

    knn_idx_l = []
    for lvl in range(1, 5):
        idx, _ = _knn(pts[lvl], ptsT[lvl - 1], KNN_K, want_d2=False)
        knn_idx_l.append(idx)
    interp = []
    for lvl in range(1, 5):
        idx, d2 = _knn(pts[lvl - 1], ptsT[lvl], 3, want_d2=True)
        interp.append((idx, d2))

    x0 = jnp.concatenate([p, x], axis=1)
    zeros32 = jnp.zeros((1, P['W_enc1'].shape[0]), jnp.float32)
    xs = [_bn_relu_lin(x0, P['W_enc1'], zeros32, _row(P['g_enc1']),
                       _row(P['be_enc1']))]
    for lvl in range(2, 6):
        i = lvl - 2
        m = sizes[i + 1]
        kidx = knn_idx_l[i]
        w = P['W_enc%d' % lvl]
        din = w.shape[1]
        dpad = ((din + 15) // 16) * 16
        table = _pad_cols(jnp.concatenate([pts[i], xs[-1]], axis=1), dpad)
        idx_flat = kidx.T.reshape(-1)
        G3 = _sc_gather(table, idx_flat).reshape(KNN_K, m, dpad)
        xs.append(_enc_level(G3, _pad_cols(pts[i + 1], dpad),
                             _pad_cols(w, dpad), _row(P['g_enc%d' % lvl]),
                             _row(P['be_enc%d' % lvl]), KNN_K, m))

    x5 = xs[4]
    xb = pl.pallas_call(
        _mid_body,
        out_shape=jax.ShapeDtypeStruct((sizes[4], 512), jnp.float32),
    )(x5, P['Wl2_5'], _row(P['bl2_5']), P['Wl1_5'][:, :512],
      P['Wl1_5'][:, 512:], _row(P['bl1_5']), _row(P['gl1_5']),
      _row(P['bel1_5']))

    for lvl in range(4, 0, -1):
        mf = sizes[lvl - 1]
        xf = xs[lvl - 1]
        cf = xf.shape[1]
        a = _bn_relu_lin(xf, P['Wl1_%d' % lvl], _row(P['bl1_%d' % lvl]),
                         _row(P['gl1_%d' % lvl]), _row(P['bel1_%d' % lvl]))
        f2 = _bn_relu_lin(xb, P['Wl2_%d' % lvl], _row(P['bl2_%d' % lvl]),
                          _row(P['gl2_%d' % lvl]), _row(P['bel2_%d' % lvl]))
        ii, d2 = interp[lvl - 1]
        G3 = _sc_gather(f2, ii.T.reshape(-1)).reshape(3, mf, cf)
        xb = _dec_combine(a, G3, d2)

    h1, s1, q1 = _lin_stats(xb, P['Wd1'], _row(P['bd1']))
    f = _norm_head(h1, s1, q1, _row(P['gd']), _row(P['bed']),
                   P['Wd2'], _row(P['bd2']))
    h2, s2, q2 = _lin_stats(f, P['Wb1'], _row(P['bb1']))
    out = _norm_head(h2, s2, q2, _row(P['gb']), _row(P['beb']),
                     P['Wb2'], _row(P['bb2']))
    return out
